# Initial kernel scaffold; baseline (speedup 1.0000x reference)
import jax
import jax.numpy as jnp
from jax.experimental import pallas as pl


def kernel(subtoken_idxs, lengths, embedding_table, dense_kernel):
    raise NotImplementedError("write your pallas kernel here")


# speedup vs baseline: 1.6830x; 1.6830x over previous
import functools

import jax
import jax.numpy as jnp
from jax import lax
from jax.experimental import pallas as pl
from jax.experimental.pallas import tpu as pltpu
from jax.experimental.pallas import tpu_sc as plsc

B = 4096
L = 20
E = 128
LANES = 16
NC = 2
NS = 16
NW = NC * NS
BPW = B // NW
CB = 4
CIDX = CB * L
NCHUNK = BPW // CB

_mesh = plsc.VectorSubcoreMesh(core_axis_name="c", subcore_axis_name="s")


@functools.partial(
    pl.kernel,
    out_type=jax.ShapeDtypeStruct((B, E), jnp.float32),
    mesh=_mesh,
    scratch_types=[
        pltpu.VMEM((BPW * L,), jnp.int32),
        pltpu.VMEM((BPW * LANES,), jnp.float32),
        pltpu.VMEM((CIDX, E), jnp.float32),
        pltpu.VMEM((CB, E), jnp.float32),
        pltpu.SemaphoreType.DMA,
    ],
)
def _sc_pool(idx_hbm, lenf_hbm, table_hbm, out_hbm,
             idx_v, lenf_v, rows_v, stage_v, sem):
    wid = lax.axis_index("s") * NC + lax.axis_index("c")
    base = wid * BPW

    pltpu.sync_copy(idx_hbm.at[pl.ds(base * L, BPW * L)], idx_v)
    pltpu.sync_copy(lenf_hbm.at[pl.ds(base * LANES, BPW * LANES)], lenf_v)

    def chunk_body(c, carry):
        cp = pltpu.async_copy(
            table_hbm.at[idx_v.at[pl.ds(c * CIDX, CIDX)]], rows_v, sem)
        cp.wait()
        for j in range(CB):
            b = c * CB + j
            lens = lenf_v[pl.ds(b * LANES, LANES)]
            recs = 1.0 / (lens + 1e-10)
            acc = [jnp.zeros((LANES,), jnp.float32) for _ in range(E // LANES)]
            for l in range(L):
                m = jnp.full((LANES,), float(l), jnp.float32) < lens
                for e in range(E // LANES):
                    v = rows_v[j * L + l, pl.ds(e * LANES, LANES)]
                    acc[e] = acc[e] + jnp.where(m, v, 0.0)
            for e in range(E // LANES):
                stage_v[j, pl.ds(e * LANES, LANES)] = acc[e] * recs
        pltpu.sync_copy(stage_v, out_hbm.at[pl.ds(base + c * CB, CB)])
        return carry

    lax.fori_loop(0, NCHUNK, chunk_body, 0)


def _tc_matmul_body(x_ref, w_ref, o_ref):
    o_ref[...] = jnp.dot(x_ref[...], w_ref[...],
                         preferred_element_type=jnp.float32)


def _tc_matmul(x, w):
    blk = 512
    return pl.pallas_call(
        _tc_matmul_body,
        out_shape=jax.ShapeDtypeStruct((B, E), jnp.float32),
        grid=(B // blk,),
        in_specs=[
            pl.BlockSpec((blk, E), lambda i: (i, 0)),
            pl.BlockSpec((E, E), lambda i: (0, 0)),
        ],
        out_specs=pl.BlockSpec((blk, E), lambda i: (i, 0)),
    )(x, w)


def kernel(subtoken_idxs, lengths, embedding_table, dense_kernel):
    idx_flat = subtoken_idxs.astype(jnp.int32).reshape(B * L)
    lenf_rep = jnp.broadcast_to(
        lengths.astype(jnp.float32)[:, None], (B, LANES)).reshape(B * LANES)
    pooled = _sc_pool(idx_flat, lenf_rep, embedding_table)
    return _tc_matmul(pooled, dense_kernel)
